# Initial kernel scaffold; baseline (speedup 1.0000x reference)
import jax
import jax.numpy as jnp
from jax.experimental import pallas as pl


def kernel(hidden_states, weight, e_score_correction_bias):
    raise NotImplementedError("write your pallas kernel here")


# speedup vs baseline: 5.0514x; 5.0514x over previous
import functools

import jax
import jax.numpy as jnp
from jax.experimental import pallas as pl

NUM_EXPERTS = 64
TOP_K = 8
N_GROUP = 8
TOPK_GROUP = 4
EPG = NUM_EXPERTS // N_GROUP
ROUTED_SCALING_FACTOR = 2.5

TOKEN_BLOCK = 256
NEG = -1e30


def _router_kernel(hs_ref, wt_ref, bias_ref, idx_ref, w_ref):
    x = hs_ref[...]
    wt = wt_ref[...]
    logits = jax.lax.dot_general(
        x.astype(jnp.bfloat16), wt.astype(jnp.bfloat16),
        (((1,), (0,)), ((), ())),
        preferred_element_type=jnp.float32)
    scores = jax.nn.sigmoid(logits)
    sfc = scores + bias_ref[...]

    T = sfc.shape[0]
    lane = jax.lax.broadcasted_iota(jnp.int32, (T, NUM_EXPERTS), 1)
    gid = lane // EPG
    off = lane % EPG

    r0 = jax.lax.broadcasted_iota(jnp.int32, (NUM_EXPERTS, NUM_EXPERTS), 0) // EPG
    r1 = jax.lax.broadcasted_iota(jnp.int32, (NUM_EXPERTS, NUM_EXPERTS), 1) // EPG
    gg = (r0 == r1).astype(jnp.float32)
    gsum = jax.lax.dot_general(
        sfc, gg, (((1,), (0,)), ((), ())),
        precision=jax.lax.Precision.HIGHEST,
        preferred_element_type=jnp.float32)

    grank = jnp.zeros((T, NUM_EXPERTS), jnp.int32)
    for j in range(N_GROUP):
        gj = gsum[:, j * EPG:j * EPG + 1]
        beats = (gj > gsum) | ((gj == gsum) & (j < gid))
        grank = grank + beats.astype(jnp.int32)

    selected = grank < TOPK_GROUP
    p = grank * EPG + off
    cand = jnp.where(selected, sfc, NEG)

    taken = jnp.zeros((T, NUM_EXPERTS), jnp.bool_)
    idx_cols = []
    w_cols = []
    for _ in range(TOP_K):
        avail = jnp.where(taken, NEG, cand)
        m = jnp.max(avail, axis=1, keepdims=True)
        at_max = avail == m
        pmin = jnp.min(jnp.where(at_max, p, jnp.int32(2 ** 30)),
                       axis=1, keepdims=True)
        chosen = at_max & (p == pmin)
        idx_cols.append(jnp.sum(jnp.where(chosen, lane, 0),
                                axis=1, keepdims=True))
        w_cols.append(jnp.sum(jnp.where(chosen, scores, 0.0),
                              axis=1, keepdims=True))
        taken = taken | chosen

    topk_idx = jnp.concatenate(idx_cols, axis=1)
    topk_w = jnp.concatenate(w_cols, axis=1)
    topk_w = topk_w / (jnp.sum(topk_w, axis=1, keepdims=True) + 1e-20)
    topk_w = topk_w * ROUTED_SCALING_FACTOR

    idx_ref[...] = topk_idx
    w_ref[...] = topk_w


@functools.partial(jax.jit, static_argnames=())
def kernel(hidden_states, weight, e_score_correction_bias):
    hs = hidden_states.reshape(-1, hidden_states.shape[-1]).astype(jnp.float32)
    n_tokens, hidden = hs.shape
    wt = weight.astype(jnp.float32).T
    bias = e_score_correction_bias.astype(jnp.float32).reshape(1, NUM_EXPERTS)

    grid = (n_tokens // TOKEN_BLOCK,)
    topk_idx, topk_w = pl.pallas_call(
        _router_kernel,
        grid=grid,
        in_specs=[
            pl.BlockSpec((TOKEN_BLOCK, hidden), lambda i: (i, 0)),
            pl.BlockSpec((hidden, NUM_EXPERTS), lambda i: (0, 0)),
            pl.BlockSpec((1, NUM_EXPERTS), lambda i: (0, 0)),
        ],
        out_specs=[
            pl.BlockSpec((TOKEN_BLOCK, TOP_K), lambda i: (i, 0)),
            pl.BlockSpec((TOKEN_BLOCK, TOP_K), lambda i: (i, 0)),
        ],
        out_shape=[
            jax.ShapeDtypeStruct((n_tokens, TOP_K), jnp.int32),
            jax.ShapeDtypeStruct((n_tokens, TOP_K), jnp.float32),
        ],
    )(hs, wt, bias)
    return (topk_idx, topk_w)


# speedup vs baseline: 5.3615x; 1.0614x over previous
import functools

import jax
import jax.numpy as jnp
from jax.experimental import pallas as pl

NUM_EXPERTS = 64
TOP_K = 8
N_GROUP = 8
TOPK_GROUP = 4
EPG = NUM_EXPERTS // N_GROUP
ROUTED_SCALING_FACTOR = 2.5

TOKEN_BLOCK = 256
NEG = -1e30


def _router_kernel(hs_ref, wt_ref, bias_ref, idx_ref, w_ref):
    x = hs_ref[...]
    wt = wt_ref[...]
    logits = jax.lax.dot_general(
        x.astype(jnp.bfloat16), wt.astype(jnp.bfloat16),
        (((1,), (0,)), ((), ())),
        preferred_element_type=jnp.float32)
    scores = jax.nn.sigmoid(logits)
    sfc = scores + bias_ref[...]

    T = sfc.shape[0]
    lane = jax.lax.broadcasted_iota(
        jnp.int32, (T, NUM_EXPERTS), 1).astype(jnp.float32)
    gidf = jnp.floor(lane * (1.0 / EPG)) * EPG

    r0 = jax.lax.broadcasted_iota(jnp.int32, (NUM_EXPERTS, NUM_EXPERTS), 0) // EPG
    r1 = jax.lax.broadcasted_iota(jnp.int32, (NUM_EXPERTS, NUM_EXPERTS), 1) // EPG
    gg = (r0 == r1).astype(jnp.float32)
    gsum = jax.lax.dot_general(
        sfc, gg, (((1,), (0,)), ((), ())),
        precision=jax.lax.Precision.HIGHEST,
        preferred_element_type=jnp.float32)

    grank = jnp.zeros((T, NUM_EXPERTS), jnp.float32)
    for j in range(N_GROUP):
        gj = gsum[:, j * EPG:j * EPG + 1]
        beats = (gj > gsum) | ((gj == gsum) & (j * EPG < gidf))
        grank = grank + beats.astype(jnp.float32)

    selected = grank < TOPK_GROUP
    p = grank * 512.0 + lane
    avail = jnp.where(selected, sfc, NEG)

    m_cols = []
    e_cols = []
    for _ in range(TOP_K):
        m = jnp.max(avail, axis=1, keepdims=True)
        pm = jnp.min(jnp.where(avail == m, p, 4096.0),
                     axis=1, keepdims=True)
        m_cols.append(m)
        e_cols.append(pm - jnp.floor(pm * (1.0 / 512.0)) * 512.0)
        avail = jnp.where(p == pm, NEG, avail)

    topk_val = jnp.concatenate(m_cols, axis=1)
    topk_e = jnp.concatenate(e_cols, axis=1)

    bias_at = jnp.zeros((T, TOP_K), jnp.float32)
    for e in range(NUM_EXPERTS):
        bias_at = bias_at + jnp.where(topk_e == float(e), bias_ref[0, e], 0.0)
    topk_w = topk_val - bias_at
    topk_w = topk_w / (jnp.sum(topk_w, axis=1, keepdims=True) + 1e-20)
    topk_w = topk_w * ROUTED_SCALING_FACTOR

    idx_ref[...] = topk_e.astype(jnp.int32)
    w_ref[...] = topk_w


@functools.partial(jax.jit, static_argnames=())
def kernel(hidden_states, weight, e_score_correction_bias):
    hs = hidden_states.reshape(-1, hidden_states.shape[-1]).astype(jnp.float32)
    n_tokens, hidden = hs.shape
    wt = weight.astype(jnp.float32).T
    bias = e_score_correction_bias.astype(jnp.float32).reshape(1, NUM_EXPERTS)

    grid = (n_tokens // TOKEN_BLOCK,)
    topk_idx, topk_w = pl.pallas_call(
        _router_kernel,
        grid=grid,
        in_specs=[
            pl.BlockSpec((TOKEN_BLOCK, hidden), lambda i: (i, 0)),
            pl.BlockSpec((hidden, NUM_EXPERTS), lambda i: (0, 0)),
            pl.BlockSpec((1, NUM_EXPERTS), lambda i: (0, 0)),
        ],
        out_specs=[
            pl.BlockSpec((TOKEN_BLOCK, TOP_K), lambda i: (i, 0)),
            pl.BlockSpec((TOKEN_BLOCK, TOP_K), lambda i: (i, 0)),
        ],
        out_shape=[
            jax.ShapeDtypeStruct((n_tokens, TOP_K), jnp.int32),
            jax.ShapeDtypeStruct((n_tokens, TOP_K), jnp.float32),
        ],
    )(hs, wt, bias)
    return (topk_idx, topk_w)


# speedup vs baseline: 6.6136x; 1.2335x over previous
import functools

import jax
import jax.numpy as jnp
from jax.experimental import pallas as pl

NUM_EXPERTS = 64
TOP_K = 8
N_GROUP = 8
TOPK_GROUP = 4
EPG = NUM_EXPERTS // N_GROUP
ROUTED_SCALING_FACTOR = 2.5

TOKEN_BLOCK = 256
NEG = -1e30


def _router_kernel(hs_ref, wt_ref, bias_ref, idx_ref, w_ref):
    x = hs_ref[...]
    wt = wt_ref[...]
    logits = jax.lax.dot_general(
        x.astype(jnp.bfloat16), wt.astype(jnp.bfloat16),
        (((1,), (0,)), ((), ())),
        preferred_element_type=jnp.float32)
    scores = jax.nn.sigmoid(logits)
    sfc = scores + bias_ref[...]

    T = sfc.shape[0]
    lane = jax.lax.broadcasted_iota(
        jnp.int32, (T, NUM_EXPERTS), 1).astype(jnp.float32)
    gidf = jnp.floor(lane * (1.0 / EPG)) * EPG

    r0 = jax.lax.broadcasted_iota(jnp.int32, (NUM_EXPERTS, NUM_EXPERTS), 0) // EPG
    r1 = jax.lax.broadcasted_iota(jnp.int32, (NUM_EXPERTS, NUM_EXPERTS), 1) // EPG
    gg = (r0 == r1).astype(jnp.float32)
    gsum = jax.lax.dot_general(
        sfc, gg, (((1,), (0,)), ((), ())),
        precision=jax.lax.Precision.HIGHEST,
        preferred_element_type=jnp.float32)

    grank = jnp.zeros((T, NUM_EXPERTS), jnp.float32)
    for j in range(N_GROUP):
        gj = gsum[:, j * EPG:j * EPG + 1]
        beats = (gj > gsum) | ((gj == gsum) & (j * EPG < gidf))
        grank = grank + beats.astype(jnp.float32)

    selected = grank < TOPK_GROUP
    p = grank * 512.0 + lane
    avail = jnp.where(selected, sfc, NEG)

    e_cols = []
    w_cols = []
    for _ in range(TOP_K):
        m = jnp.max(avail, axis=1, keepdims=True)
        pm = jnp.min(jnp.where(avail == m, p, 4096.0),
                     axis=1, keepdims=True)
        chosen = p == pm
        e_cols.append(pm - jnp.floor(pm * (1.0 / 512.0)) * 512.0)
        w_cols.append(jnp.sum(jnp.where(chosen, scores, 0.0),
                              axis=1, keepdims=True))
        avail = jnp.where(chosen, NEG, avail)

    topk_e = jnp.concatenate(e_cols, axis=1)
    topk_w = jnp.concatenate(w_cols, axis=1)
    topk_w = topk_w / (jnp.sum(topk_w, axis=1, keepdims=True) + 1e-20)
    topk_w = topk_w * ROUTED_SCALING_FACTOR

    idx_ref[...] = topk_e.astype(jnp.int32)
    w_ref[...] = topk_w


@functools.partial(jax.jit, static_argnames=())
def kernel(hidden_states, weight, e_score_correction_bias):
    hs = hidden_states.reshape(-1, hidden_states.shape[-1]).astype(jnp.float32)
    n_tokens, hidden = hs.shape
    wt = weight.astype(jnp.float32).T
    bias = e_score_correction_bias.astype(jnp.float32).reshape(1, NUM_EXPERTS)

    grid = (n_tokens // TOKEN_BLOCK,)
    topk_idx, topk_w = pl.pallas_call(
        _router_kernel,
        grid=grid,
        in_specs=[
            pl.BlockSpec((TOKEN_BLOCK, hidden), lambda i: (i, 0)),
            pl.BlockSpec((hidden, NUM_EXPERTS), lambda i: (0, 0)),
            pl.BlockSpec((1, NUM_EXPERTS), lambda i: (0, 0)),
        ],
        out_specs=[
            pl.BlockSpec((TOKEN_BLOCK, TOP_K), lambda i: (i, 0)),
            pl.BlockSpec((TOKEN_BLOCK, TOP_K), lambda i: (i, 0)),
        ],
        out_shape=[
            jax.ShapeDtypeStruct((n_tokens, TOP_K), jnp.int32),
            jax.ShapeDtypeStruct((n_tokens, TOP_K), jnp.float32),
        ],
    )(hs, wt, bias)
    return (topk_idx, topk_w)


# speedup vs baseline: 8.0385x; 1.2155x over previous
import functools

import jax
import jax.numpy as jnp
from jax.experimental import pallas as pl

NUM_EXPERTS = 64
TOP_K = 8
N_GROUP = 8
TOPK_GROUP = 4
EPG = NUM_EXPERTS // N_GROUP
ROUTED_SCALING_FACTOR = 2.5

TOKEN_BLOCK = 512
NEG = -1e30


def _router_kernel(hs_ref, wt_ref, bias_ref, idx_ref, w_ref):
    x = hs_ref[...]
    wt = wt_ref[...]
    logits = jax.lax.dot_general(
        x.astype(jnp.bfloat16), wt.astype(jnp.bfloat16),
        (((1,), (0,)), ((), ())),
        preferred_element_type=jnp.float32)
    scores = jax.nn.sigmoid(logits)
    sfc = scores + bias_ref[...]

    T = sfc.shape[0]
    lane = jax.lax.broadcasted_iota(
        jnp.int32, (T, NUM_EXPERTS), 1).astype(jnp.float32)
    gidf = jnp.floor(lane * (1.0 / EPG)) * EPG

    r0 = jax.lax.broadcasted_iota(jnp.int32, (NUM_EXPERTS, NUM_EXPERTS), 0) // EPG
    r1 = jax.lax.broadcasted_iota(jnp.int32, (NUM_EXPERTS, NUM_EXPERTS), 1) // EPG
    gg = (r0 == r1).astype(jnp.float32)
    gsum = jax.lax.dot_general(
        sfc, gg, (((1,), (0,)), ((), ())),
        precision=jax.lax.Precision.HIGHEST,
        preferred_element_type=jnp.float32)

    grank = jnp.zeros((T, NUM_EXPERTS), jnp.float32)
    for j in range(N_GROUP):
        gj = gsum[:, j * EPG:j * EPG + 1]
        beats = (gj > gsum) | ((gj == gsum) & (j * EPG < gidf))
        grank = grank + beats.astype(jnp.float32)

    selected = grank < TOPK_GROUP
    p = grank * 512.0 + lane
    avail = jnp.where(selected, sfc, NEG)

    e_cols = []
    w_cols = []
    for _ in range(TOP_K):
        m = jnp.max(avail, axis=1, keepdims=True)
        pm = jnp.min(jnp.where(avail == m, p, 4096.0),
                     axis=1, keepdims=True)
        chosen = p == pm
        e_cols.append(pm - jnp.floor(pm * (1.0 / 512.0)) * 512.0)
        w_cols.append(jnp.sum(jnp.where(chosen, scores, 0.0),
                              axis=1, keepdims=True))
        avail = jnp.where(chosen, NEG, avail)

    topk_e = jnp.concatenate(e_cols, axis=1)
    topk_w = jnp.concatenate(w_cols, axis=1)
    topk_w = topk_w / (jnp.sum(topk_w, axis=1, keepdims=True) + 1e-20)
    topk_w = topk_w * ROUTED_SCALING_FACTOR

    idx_ref[...] = topk_e.astype(jnp.int32)
    w_ref[...] = topk_w


@functools.partial(jax.jit, static_argnames=())
def kernel(hidden_states, weight, e_score_correction_bias):
    hs = hidden_states.reshape(-1, hidden_states.shape[-1]).astype(jnp.float32)
    n_tokens, hidden = hs.shape
    wt = weight.astype(jnp.float32).T
    bias = e_score_correction_bias.astype(jnp.float32).reshape(1, NUM_EXPERTS)

    grid = (n_tokens // TOKEN_BLOCK,)
    topk_idx, topk_w = pl.pallas_call(
        _router_kernel,
        grid=grid,
        in_specs=[
            pl.BlockSpec((TOKEN_BLOCK, hidden), lambda i: (i, 0)),
            pl.BlockSpec((hidden, NUM_EXPERTS), lambda i: (0, 0)),
            pl.BlockSpec((1, NUM_EXPERTS), lambda i: (0, 0)),
        ],
        out_specs=[
            pl.BlockSpec((TOKEN_BLOCK, TOP_K), lambda i: (i, 0)),
            pl.BlockSpec((TOKEN_BLOCK, TOP_K), lambda i: (i, 0)),
        ],
        out_shape=[
            jax.ShapeDtypeStruct((n_tokens, TOP_K), jnp.int32),
            jax.ShapeDtypeStruct((n_tokens, TOP_K), jnp.float32),
        ],
    )(hs, wt, bias)
    return (topk_idx, topk_w)
